# Initial kernel scaffold; baseline (speedup 1.0000x reference)
import jax
import jax.numpy as jnp
from jax.experimental import pallas as pl


def kernel(out_p3, out_p4, out_p5, out_p6, out_p7, anc_p3, anc_p4, anc_p5, anc_p6, anc_p7):
    raise NotImplementedError("write your pallas kernel here")


# speedup vs baseline: 1.2322x; 1.2322x over previous
import math

import jax
import jax.numpy as jnp
from jax.experimental import pallas as pl

_NUM_CLASSES = 80
_SCORE_THRESH = 0.05
_TOPK = 1000
_NMS_THRESH = 0.5
_MAX_DET = 100
_SCALE_CLAMP = math.log(1000.0 / 16.0)
_LANES = 128


def _decode_nms_kernel(dx_ref, dy_ref, dw_ref, dh_ref,
                       ax1_ref, ay1_ref, ax2_ref, ay2_ref,
                       sc_ref, cl_ref, det_ref, cls_ref):
    aw = ax2_ref[...] - ax1_ref[...]
    ah = ay2_ref[...] - ay1_ref[...]
    cx = ax1_ref[...] + 0.5 * aw
    cy = ay1_ref[...] + 0.5 * ah
    dw = jnp.minimum(dw_ref[...], _SCALE_CLAMP)
    dh = jnp.minimum(dh_ref[...], _SCALE_CLAMP)
    pcx = dx_ref[...] * aw + cx
    pcy = dy_ref[...] * ah + cy
    pw = jnp.exp(dw) * aw
    ph = jnp.exp(dh) * ah
    x1 = pcx - 0.5 * pw
    y1 = pcy - 0.5 * ph
    x2 = pcx + 0.5 * pw
    y2 = pcy + 0.5 * ph

    sc = sc_ref[...]
    cls_i = cl_ref[...]
    shape = sc.shape
    areas = jnp.maximum(x2 - x1, 0.0) * jnp.maximum(y2 - y1, 0.0)
    s = jnp.where(sc > _SCORE_THRESH, sc, -jnp.inf)
    flat_iota = (jax.lax.broadcasted_iota(jnp.int32, shape, 0) * _LANES
                 + jax.lax.broadcasted_iota(jnp.int32, shape, 1))

    def body(i, cand):
        maxval = jnp.max(cand)
        ok = maxval > -jnp.inf
        pick = jnp.min(jnp.where(cand == maxval, flat_iota, jnp.int32(1 << 30)))
        m = flat_iota == pick

        x1p = jnp.sum(jnp.where(m, x1, 0.0))
        y1p = jnp.sum(jnp.where(m, y1, 0.0))
        x2p = jnp.sum(jnp.where(m, x2, 0.0))
        y2p = jnp.sum(jnp.where(m, y2, 0.0))
        spk = jnp.sum(jnp.where(m, sc, 0.0))
        cpk = jnp.sum(jnp.where(m, cls_i, 0))

        xx1 = jnp.maximum(x1p, x1)
        yy1 = jnp.maximum(y1p, y1)
        xx2 = jnp.minimum(x2p, x2)
        yy2 = jnp.minimum(y2p, y2)
        inter = jnp.maximum(xx2 - xx1, 0.0) * jnp.maximum(yy2 - yy1, 0.0)
        area_p = (jnp.maximum(x2p - x1p, 0.0)
                  * jnp.maximum(y2p - y1p, 0.0))
        iou = inter / (area_p + areas - inter + 1e-9)
        sup = (cls_i == cpk) & (iou > _NMS_THRESH)
        cand = jnp.where(ok & sup, -jnp.inf, cand)

        row = jnp.concatenate(
            [x1p.reshape(1, 1), y1p.reshape(1, 1), x2p.reshape(1, 1),
             y2p.reshape(1, 1), spk.reshape(1, 1)], axis=1)
        det_ref[pl.ds(i, 1), :] = jnp.where(ok, row, 0.0)
        cls_ref[pl.ds(i, 1), :] = jnp.where(ok, cpk, -1).reshape(1, 1)
        return cand

    jax.lax.fori_loop(0, _MAX_DET, body, s)


def kernel(out_p3, out_p4, out_p5, out_p6, out_p7,
           anc_p3, anc_p4, anc_p5, anc_p6, anc_p7):
    outs = [out_p3, out_p4, out_p5, out_p6, out_p7]
    ancs = [anc_p3, anc_p4, anc_p5, anc_p6, anc_p7]

    d_all, a_all, s_all, c_all = [], [], [], []
    for o, a in zip(outs, ancs):
        cls_flat = o[0, :, 4:].reshape(-1)
        reg = o[0, :, :4]
        k = min(_TOPK, reg.shape[0])
        prob, idxs = jax.lax.top_k(cls_flat, k)
        a_idx = idxs // _NUM_CLASSES
        c_idx = idxs % _NUM_CLASSES
        d_all.append(reg[a_idx])
        a_all.append(a[a_idx])
        s_all.append(prob)
        c_all.append(c_idx)

    deltas = jnp.concatenate(d_all)
    anchors = jnp.concatenate(a_all)
    scores = jnp.concatenate(s_all)
    classes = jnp.concatenate(c_all).astype(jnp.int32)

    n = deltas.shape[0]
    rows = -(-n // _LANES)
    pad = rows * _LANES - n
    deltas = jnp.concatenate([deltas, jnp.zeros((pad, 4), jnp.float32)])
    anchors = jnp.concatenate(
        [anchors,
         jnp.tile(jnp.array([[0.0, 0.0, 1.0, 1.0]], jnp.float32), (pad, 1))])
    scores = jnp.concatenate([scores, jnp.full((pad,), -jnp.inf, jnp.float32)])
    classes = jnp.concatenate([classes, jnp.zeros((pad,), jnp.int32)])

    comps = [deltas[:, j].reshape(rows, _LANES) for j in range(4)]
    comps += [anchors[:, j].reshape(rows, _LANES) for j in range(4)]
    comps += [scores.reshape(rows, _LANES), classes.reshape(rows, _LANES)]

    det, cls_out = pl.pallas_call(
        _decode_nms_kernel,
        out_shape=[
            jax.ShapeDtypeStruct((_MAX_DET, 5), jnp.float32),
            jax.ShapeDtypeStruct((_MAX_DET, 1), jnp.int32),
        ],
    )(*comps)
    return det, cls_out[:, 0]
